# Initial kernel scaffold; baseline (speedup 1.0000x reference)
import functools

import jax
import jax.numpy as jnp
from jax import lax
from jax.experimental import pallas as pl
from jax.experimental.pallas import tpu as pltpu
from jax.experimental.pallas import tpu_sc as plsc

VOCAB = 100000
D = 64
BATCH = 4096
SEQ = 50
LANES = 16

NC = 2
NS = 16
NW = NC * NS

TOTAL = BATCH * SEQ
PER_W = TOTAL // NW
DMA_ROWS = 100
CHUNK = 400
Q = CHUNK // DMA_ROWS
NCHUNK = PER_W // CHUNK
IDX_ROWS_PER_W = PER_W // DMA_ROWS
NBLK = CHUNK // SEQ


def _pos_table():
    pos = jnp.arange(SEQ, dtype=jnp.float32)[:, None]
    i = jnp.arange(D, dtype=jnp.float32)[None, :]
    angle = pos / jnp.power(10000.0, 2.0 * jnp.floor(i / 2.0) / D)
    angle = angle.at[:, 0::2].set(jnp.sin(angle[:, 0::2]))
    angle = angle.at[:, 1::2].set(jnp.cos(angle[:, 1::2]))
    return angle


def _sc_embed(weight, idx2d, pos):
    mesh = plsc.VectorSubcoreMesh(core_axis_name="c", subcore_axis_name="s")

    @functools.partial(
        pl.kernel,
        mesh=mesh,
        out_type=jax.ShapeDtypeStruct((NW * NCHUNK, CHUNK, D), jnp.float32),
        scratch_types=[
            pltpu.VMEM((IDX_ROWS_PER_W, DMA_ROWS), jnp.int32),
            pltpu.VMEM((SEQ, D), jnp.float32),
            pltpu.VMEM((CHUNK, D), jnp.float32),
            pltpu.SemaphoreType.DMA,
        ],
    )
    def k(w_hbm, idx_hbm, pos_hbm, out_hbm, idx_v, pos_v, gbuf, gsem):
        wid = lax.axis_index("s") * NC + lax.axis_index("c")
        pltpu.sync_copy(idx_hbm.at[pl.ds(wid * IDX_ROWS_PER_W, IDX_ROWS_PER_W)],
                        idx_v)
        pltpu.sync_copy(pos_hbm, pos_v)

        def chunk_body(c, carry):
            copies = []
            for q in range(Q):
                copies.append(
                    pltpu.async_copy(
                        w_hbm.at[idx_v.at[c * Q + q]],
                        gbuf.at[pl.ds(q * DMA_ROWS, DMA_ROWS)],
                        gsem,
                    ))
            for cp in copies:
                cp.wait()

            def sblk(sb, carry2):
                for r in range(SEQ):
                    row = sb * SEQ + r
                    for j in range(D // LANES):
                        sl = pl.ds(j * LANES, LANES)
                        gbuf[row, sl] = gbuf[row, sl] * 0.125 + pos_v[r, sl]
                return carry2

            lax.fori_loop(0, NBLK, sblk, 0)
            pltpu.sync_copy(gbuf, out_hbm.at[wid * NCHUNK + c])
            return carry

        lax.fori_loop(0, NCHUNK, chunk_body, 0)

    return k(weight, idx2d, pos)


def kernel(input, weight):
    idx2d = input.reshape(TOTAL // DMA_ROWS, DMA_ROWS)
    pos = _pos_table()
    out = _sc_embed(weight, idx2d, pos)
    return out.reshape(BATCH, SEQ, D)


# speedup vs baseline: 3.0808x; 3.0808x over previous
import functools

import jax
import jax.numpy as jnp
from jax import lax
from jax.experimental import pallas as pl
from jax.experimental.pallas import tpu as pltpu
from jax.experimental.pallas import tpu_sc as plsc

VOCAB = 100000
D = 64
BATCH = 4096
SEQ = 50
LANES = 16

NC = 2
NS = 16
NW = NC * NS

TOTAL = BATCH * SEQ
PER_W = TOTAL // NW
DMA_ROWS = 100
CHUNK = 400
Q = CHUNK // DMA_ROWS
NCHUNK = PER_W // CHUNK
IDX_ROWS_PER_W = PER_W // DMA_ROWS
NBLK = CHUNK // SEQ


def _pos_table():
    pos = jnp.arange(SEQ, dtype=jnp.float32)[:, None]
    i = jnp.arange(D, dtype=jnp.float32)[None, :]
    angle = pos / jnp.power(10000.0, 2.0 * jnp.floor(i / 2.0) / D)
    angle = angle.at[:, 0::2].set(jnp.sin(angle[:, 0::2]))
    angle = angle.at[:, 1::2].set(jnp.cos(angle[:, 1::2]))
    return angle


def _sc_embed(weight, idx2d, pos):
    mesh = plsc.VectorSubcoreMesh(core_axis_name="c", subcore_axis_name="s")

    @functools.partial(
        pl.kernel,
        mesh=mesh,
        compiler_params=pltpu.CompilerParams(use_tc_tiling_on_sc=False),
        out_type=jax.ShapeDtypeStruct((NW * NCHUNK, CHUNK, D), jnp.float32),
        scratch_types=[
            pltpu.VMEM((IDX_ROWS_PER_W, DMA_ROWS), jnp.int32),
            pltpu.VMEM((SEQ, D), jnp.float32),
            pltpu.VMEM((CHUNK, D), jnp.float32),
            pltpu.SemaphoreType.DMA,
        ],
    )
    def k(w_hbm, idx_hbm, pos_hbm, out_hbm, idx_v, pos_v, gbuf, gsem):
        wid = lax.axis_index("s") * NC + lax.axis_index("c")
        pltpu.sync_copy(idx_hbm.at[pl.ds(wid * IDX_ROWS_PER_W, IDX_ROWS_PER_W)],
                        idx_v)
        pltpu.sync_copy(pos_hbm, pos_v)

        def chunk_body(c, carry):
            copies = []
            for q in range(Q):
                copies.append(
                    pltpu.async_copy(
                        w_hbm.at[idx_v.at[c * Q + q]],
                        gbuf.at[pl.ds(q * DMA_ROWS, DMA_ROWS)],
                        gsem,
                    ))
            for cp in copies:
                cp.wait()

            def sblk(sb, carry2):
                for r in range(SEQ):
                    row = sb * SEQ + r
                    for j in range(D // LANES):
                        sl = pl.ds(j * LANES, LANES)
                        gbuf[row, sl] = gbuf[row, sl] * 0.125 + pos_v[r, sl]
                return carry2

            lax.fori_loop(0, NBLK, sblk, 0)
            pltpu.sync_copy(gbuf, out_hbm.at[wid * NCHUNK + c])
            return carry

        lax.fori_loop(0, NCHUNK, chunk_body, 0)

    return k(weight, idx2d, pos)


def kernel(input, weight):
    idx2d = input.reshape(TOTAL // DMA_ROWS, DMA_ROWS)
    pos = _pos_table()
    out = _sc_embed(weight, idx2d, pos)
    return out.reshape(BATCH, SEQ, D)
